# baseline (speedup 1.0000x reference)
import functools

import jax
import jax.numpy as jnp
from jax import lax
from jax.experimental import pallas as pl
from jax.experimental.pallas import tpu as pltpu
from jax.experimental.pallas import tpu_sc as plsc

B = 4096
D_DENSE = 13
F = 26
V = 100000
E = 64
D0 = E + F * E


_NC = 2
_NS = 16
_NW = _NC * _NS
_ROWS = B * F
_BPW = _ROWS // _NW
_CHUNK = 832
_NCHUNK = _BPW // _CHUNK


def _sc_gather(table, flat_idx):
    mesh = plsc.VectorSubcoreMesh(core_axis_name="c", subcore_axis_name="s")

    @functools.partial(
        pl.kernel,
        mesh=mesh,
        compiler_params=pltpu.CompilerParams(use_tc_tiling_on_sc=False),
        out_type=jax.ShapeDtypeStruct((_ROWS, E), jnp.float32),
        scratch_types=[
            pltpu.VMEM((_NCHUNK, _CHUNK), jnp.int32),
            pltpu.VMEM((_CHUNK, E), jnp.float32),
            pltpu.VMEM((_CHUNK, E), jnp.float32),
            pltpu.SemaphoreType.DMA,
            pltpu.SemaphoreType.DMA,
        ],
    )
    def gather_kernel(table_hbm, idx_hbm, out_hbm, idx_v, buf0, buf1, sem0, sem1):
        wid = lax.axis_index("s") * _NC + lax.axis_index("c")
        base = wid * _BPW
        pltpu.sync_copy(idx_hbm.at[wid], idx_v)
        bufs = (buf0, buf1)
        sems = (sem0, sem1)
        handles = [None, None]
        handles[0] = pltpu.async_copy(table_hbm.at[idx_v.at[0]], bufs[0], sems[0])
        for j in range(_NCHUNK):
            cur = j % 2
            nxt = (j + 1) % 2
            if j + 1 < _NCHUNK:
                handles[nxt] = pltpu.async_copy(
                    table_hbm.at[idx_v.at[j + 1]], bufs[nxt], sems[nxt])
            handles[cur].wait()
            pltpu.sync_copy(bufs[cur], out_hbm.at[pl.ds(base + j * _CHUNK, _CHUNK)])

    return gather_kernel(table, flat_idx)


_TILE = 256


def _dense_body(dense_ref, emb_ref,
                bw0, bb0, bw1, bb1, bw2, bb2,
                V0, U0, c0, V1, U1, c1, V2, U2, c2,
                tw0, tb0, tw1, tb1, tw2, tb2, tw3, tb3, tw4, tb4,
                out_ref):
    f32 = jnp.float32

    def mm(a, b):
        return jnp.dot(a, b, preferred_element_type=f32)

    h = jnp.maximum(mm(dense_ref[...], bw0[...]) + bb0[...], 0.0)
    h = jnp.maximum(mm(h, bw1[...]) + bb1[...], 0.0)
    dense_out = jnp.maximum(mm(h, bw2[...]) + bb2[...], 0.0)

    x0 = jnp.concatenate([dense_out, emb_ref[...]], axis=-1)
    xl = x0
    for Vm, Um, cb in ((V0, U0, c0), (V1, U1, c1), (V2, U2, c2)):
        proj = mm(xl, Vm[...])
        u = mm(proj, Um[...]) + cb[...]
        xl = x0 * u + xl

    h = jnp.maximum(mm(xl, tw0[...]) + tb0[...], 0.0)
    h = jnp.maximum(mm(h, tw1[...]) + tb1[...], 0.0)
    h = jnp.maximum(mm(h, tw2[...]) + tb2[...], 0.0)
    h = jnp.maximum(mm(h, tw3[...]) + tb3[...], 0.0)
    z = mm(h, tw4[...]) + tb4[...]
    out_ref[...] = 1.0 / (1.0 + jnp.exp(-z))


def _const_spec(shape):
    nd = len(shape)
    return pl.BlockSpec(shape, lambda i: (0,) * nd)


def _dense_chain(dense_features, emb, weights):
    (bw0, bb0, bw1, bb1, bw2, bb2,
     V0, U0, c0, V1, U1, c1, V2, U2, c2,
     tw0, tb0, tw1, tb1, tw2, tb2, tw3, tb3, tw4, tb4) = weights

    grid = (B // _TILE,)
    in_specs = [
        pl.BlockSpec((_TILE, D_DENSE), lambda i: (i, 0)),
        pl.BlockSpec((_TILE, F * E), lambda i: (i, 0)),
    ] + [_const_spec(w.shape) for w in weights]

    return pl.pallas_call(
        _dense_body,
        grid=grid,
        in_specs=in_specs,
        out_specs=pl.BlockSpec((_TILE, 1), lambda i: (i, 0)),
        out_shape=jax.ShapeDtypeStruct((B, 1), jnp.float32),
    )(dense_features, emb, *weights)


def kernel(dense_features, sparse_idx, emb_tables,
           bw0, bb0, bw1, bb1, bw2, bb2,
           V0, U0, c0, V1, U1, c1, V2, U2, c2,
           tw0, tb0, tw1, tb1, tw2, tb2, tw3, tb3, tw4, tb4):
    table = emb_tables.reshape(F * V, E)
    flat_idx = (sparse_idx + jnp.arange(F, dtype=jnp.int32)[None, :] * V)
    flat_idx = flat_idx.reshape(_NW, _NCHUNK, _CHUNK)
    rows = _sc_gather(table, flat_idx)
    return rows[:B, :1]
    emb = rows.reshape(B, F * E)

    weights = (bw0, bb0.reshape(1, -1), bw1, bb1.reshape(1, -1),
               bw2, bb2.reshape(1, -1),
               V0, U0, c0.reshape(1, -1), V1, U1, c1.reshape(1, -1),
               V2, U2, c2.reshape(1, -1),
               tw0, tb0.reshape(1, -1), tw1, tb1.reshape(1, -1),
               tw2, tb2.reshape(1, -1), tw3, tb3.reshape(1, -1),
               tw4, tb4.reshape(1, -1))
    return _dense_chain(dense_features, emb, weights)


# speedup vs baseline: 2.7534x; 2.7534x over previous
import functools

import jax
import jax.numpy as jnp
from jax import lax
from jax.experimental import pallas as pl
from jax.experimental.pallas import tpu as pltpu
from jax.experimental.pallas import tpu_sc as plsc

B = 4096
D_DENSE = 13
F = 26
V = 100000
E = 64
D0 = E + F * E


_NC = 2
_NS = 16
_NW = _NC * _NS
_TROWS = F * E
_RPW = _TROWS // _NW


def _sc_gather_t(tswap, idxT):
    mesh = plsc.VectorSubcoreMesh(core_axis_name="c", subcore_axis_name="s")

    @functools.partial(
        pl.kernel,
        mesh=mesh,
        compiler_params=pltpu.CompilerParams(use_tc_tiling_on_sc=True,
                                             needs_layout_passes=False),
        out_type=jax.ShapeDtypeStruct((_TROWS, B), jnp.float32),
        scratch_types=[
            pltpu.VMEM((V,), jnp.float32),
            pltpu.VMEM((B,), jnp.int32),
            pltpu.VMEM((B,), jnp.float32),
        ],
    )
    def gather_kernel(tbl, idx_hbm, out_hbm, row_v, idx_v, out_v):
        wid = lax.axis_index("s") * _NC + lax.axis_index("c")

        def body(k, carry):
            rid = wid * _RPW + k
            f = rid // E
            e = rid % E
            pltpu.sync_copy(idx_hbm.at[f], idx_v)
            pltpu.sync_copy(tbl.at[f, e], row_v)
            for j in range(B // 16):
                ii = idx_v[pl.ds(j * 16, 16)]
                out_v[pl.ds(j * 16, 16)] = plsc.load_gather(row_v, [ii])
            pltpu.sync_copy(out_v, out_hbm.at[rid])
            return carry

        lax.fori_loop(0, _RPW, body, 0)

    return gather_kernel(tswap, idxT)


_TILE = 256


def _mmT(w, x):
    return lax.dot_general(w, x, (((0,), (0,)), ((), ())),
                           preferred_element_type=jnp.float32)


def _dense_body_t(dT_ref, embT_ref,
                  bw0, bb0, bw1, bb1, bw2, bb2,
                  V0, U0, c0, V1, U1, c1, V2, U2, c2,
                  tw0, tb0, tw1, tb1, tw2, tb2, tw3, tb3, tw4, tb4,
                  outT_ref):
    h = jnp.maximum(_mmT(bw0[...], dT_ref[...]) + bb0[...], 0.0)
    h = jnp.maximum(_mmT(bw1[...], h) + bb1[...], 0.0)
    dh = jnp.maximum(_mmT(bw2[...], h) + bb2[...], 0.0)

    x0 = jnp.concatenate([dh, embT_ref[...]], axis=0)
    xl = x0
    for Vm, Um, cb in ((V0, U0, c0), (V1, U1, c1), (V2, U2, c2)):
        proj = _mmT(Vm[...], xl)
        u = _mmT(Um[...], proj) + cb[...]
        xl = x0 * u + xl

    h = jnp.maximum(_mmT(tw0[...], xl) + tb0[...], 0.0)
    h = jnp.maximum(_mmT(tw1[...], h) + tb1[...], 0.0)
    h = jnp.maximum(_mmT(tw2[...], h) + tb2[...], 0.0)
    h = jnp.maximum(_mmT(tw3[...], h) + tb3[...], 0.0)
    z = _mmT(tw4[...], h) + tb4[...]
    outT_ref[...] = 1.0 / (1.0 + jnp.exp(-z))


def _const_spec(shape):
    nd = len(shape)
    return pl.BlockSpec(shape, lambda i: (0,) * nd)


def _dense_chain_t(dT, embT, weights):
    grid = (B // _TILE,)
    in_specs = [
        pl.BlockSpec((D_DENSE, _TILE), lambda i: (0, i)),
        pl.BlockSpec((_TROWS, _TILE), lambda i: (0, i)),
    ] + [_const_spec(w.shape) for w in weights]

    return pl.pallas_call(
        _dense_body_t,
        grid=grid,
        in_specs=in_specs,
        out_specs=pl.BlockSpec((1, _TILE), lambda i: (0, i)),
        out_shape=jax.ShapeDtypeStruct((1, B), jnp.float32),
        compiler_params=pltpu.CompilerParams(
            vmem_limit_bytes=100 * 1024 * 1024),
    )(dT, embT, *weights)


def kernel(dense_features, sparse_idx, emb_tables,
           bw0, bb0, bw1, bb1, bw2, bb2,
           V0, U0, c0, V1, U1, c1, V2, U2, c2,
           tw0, tb0, tw1, tb1, tw2, tb2, tw3, tb3, tw4, tb4):
    tswap = jnp.swapaxes(emb_tables, 1, 2)
    idxT = sparse_idx.T
    embT = _sc_gather_t(tswap, idxT)

    weights = (bw0, bb0.reshape(-1, 1), bw1, bb1.reshape(-1, 1),
               bw2, bb2.reshape(-1, 1),
               V0, U0, c0.reshape(-1, 1), V1, U1, c1.reshape(-1, 1),
               V2, U2, c2.reshape(-1, 1),
               tw0, tb0.reshape(-1, 1), tw1, tb1.reshape(-1, 1),
               tw2, tb2.reshape(-1, 1), tw3, tb3.reshape(-1, 1),
               tw4, tb4.reshape(-1, 1))
    outT = _dense_chain_t(dense_features.T, embT, weights)
    return outT.reshape(B, 1)
